# baseline (speedup 1.0000x reference)
import jax
import jax.numpy as jnp
from jax.experimental import pallas as pl
from jax.experimental.pallas import tpu as pltpu

_N = 1024
_C = 192
_H = 8
_CN = 256
_K = 5
_ALPHA = 0.2
_SQRT_C = float(_C ** 0.5)


def _atm_body(x_ref, attn_ref, extras_ref, xm_ref, idx_ref, acc_ref):
    h = pl.program_id(1)

    @pl.when(h == 0)
    def _init():
        acc_ref[...] = attn_ref[0, 0]

    @pl.when(h > 0)
    def _accum():
        acc_ref[...] = acc_ref[...] + attn_ref[0, 0]

    @pl.when(h == _H - 1)
    def _compute():
        X = x_ref[0]
        A = acc_ref[...]

        n1 = jnp.sum(X * X, axis=1)
        g1 = jax.lax.dot_general(X, X, (((1,), (1,)), ((), ())),
                                 preferred_element_type=jnp.float32)
        d1 = jnp.sqrt(jnp.maximum(n1[:, None] + n1[None, :] - 2.0 * g1, 0.0))
        n2 = jnp.sum(A * A, axis=1)
        g2 = jax.lax.dot_general(A, A, (((1,), (1,)), ((), ())),
                                 preferred_element_type=jnp.float32)
        d2 = jnp.sqrt(jnp.maximum(n2[:, None] + n2[None, :] - 2.0 * g2, 0.0))
        dist = (1.0 - _ALPHA) * (d1 / _SQRT_C) + _ALPHA * (d2 / _SQRT_C)
        dist_max = jnp.max(dist, keepdims=True).reshape(1, 1)

        s = jnp.zeros((1, _N), jnp.float32)
        rem = jnp.full((1, _N), float(_K), jnp.float32)
        m = jnp.full((1, _N), -jnp.inf, jnp.float32)
        for _ in range(_K):
            cand = jnp.where(dist > m, dist, jnp.inf)
            m = jnp.min(cand, axis=0, keepdims=True)
            c = jnp.sum((dist == m).astype(jnp.float32), axis=0, keepdims=True)
            t = jnp.minimum(c, rem)
            s = s + jnp.where(t > 0.0, m * m * t, 0.0)
            rem = rem - t
        noise = extras_ref[0, 0:1, :]
        w_as = extras_ref[0, 1:2, :]
        dens = jnp.exp(-(s / float(_K))) + noise
        dens_col = dens.reshape(_N, 1)

        masked = jnp.where(dens_col > dens, dist, dist_max)
        dist_min = jnp.min(masked, axis=0, keepdims=True)

        score = (dist_min * dens + w_as).reshape(8, _N // 8)
        flat_n = (jax.lax.broadcasted_iota(jnp.int32, (8, _N // 8), 0)
                  * (_N // 8)
                  + jax.lax.broadcasted_iota(jnp.int32, (8, _N // 8), 1))
        flat_cn = (jax.lax.broadcasted_iota(jnp.int32, (2, _CN // 2), 0)
                   * (_CN // 2)
                   + jax.lax.broadcasted_iota(jnp.int32, (2, _CN // 2), 1))

        def topk_step(j, carry):
            sc, idxv = carry
            mx = jnp.max(sc, keepdims=True)
            amx = jnp.min(jnp.where(sc == mx, flat_n, _N), keepdims=True)
            idxv = jnp.where(flat_cn == j, amx, idxv)
            sc = jnp.where(flat_n == amx, -jnp.inf, sc)
            return sc, idxv

        _, idx_down2 = jax.lax.fori_loop(
            0, _CN, topk_step, (score, jnp.zeros((2, _CN // 2), jnp.int32)))
        idx_down = idx_down2.reshape(1, _CN)

        iota_rows = jax.lax.broadcasted_iota(jnp.int32, (_N, _CN), 0)
        iota_cols = jax.lax.broadcasted_iota(jnp.int32, (_N, _CN), 1)
        onehot = (iota_rows == idx_down).astype(jnp.float32)
        dmc = jax.lax.dot_general(dist, onehot, (((1,), (0,)), ((), ())),
                                  preferred_element_type=jnp.float32,
                                  precision=jax.lax.Precision.HIGHEST)
        mn = jnp.min(dmc, axis=1, keepdims=True)
        amn = jnp.min(jnp.where(dmc == mn, iota_cols, _CN), axis=1,
                      keepdims=True)
        is_center = jnp.sum(onehot, axis=1, keepdims=True) > 0.0
        jpos = jnp.sum(onehot * iota_cols.astype(jnp.float32), axis=1,
                       keepdims=True)
        idx_cluster = jnp.where(is_center, jpos.astype(jnp.int32), amn)

        assign = (iota_cols == idx_cluster).astype(jnp.float32)
        counts = jnp.sum(assign, axis=0, keepdims=True)
        sums = jax.lax.dot_general(assign, X, (((0,), (0,)), ((), ())),
                                   preferred_element_type=jnp.float32,
                                   precision=jax.lax.Precision.HIGHEST)
        xm_ref[0] = sums / (counts.reshape(_CN, 1) + 1e-06)
        idx_ref[0] = idx_cluster.reshape(1, _N)


def kernel(x, attn, as_out, cluster_num):
    B, N, C = x.shape
    weight = as_out.reshape(B, -1).astype(x.dtype)
    noise = jax.random.uniform(jax.random.key(1), (B, N), dtype=x.dtype) * 1e-06
    extras = jnp.stack([noise, weight], axis=1)
    xm, idx = pl.pallas_call(
        _atm_body,
        grid=(B, _H),
        in_specs=[
            pl.BlockSpec((1, N, C), lambda b, h: (b, 0, 0)),
            pl.BlockSpec((1, 1, N, N), lambda b, h: (b, h, 0, 0)),
            pl.BlockSpec((1, 2, N), lambda b, h: (b, 0, 0)),
        ],
        out_specs=[
            pl.BlockSpec((1, _CN, C), lambda b, h: (b, 0, 0)),
            pl.BlockSpec((1, 1, N), lambda b, h: (b, 0, 0)),
        ],
        out_shape=[
            jax.ShapeDtypeStruct((B, _CN, C), x.dtype),
            jax.ShapeDtypeStruct((B, 1, N), jnp.int32),
        ],
        scratch_shapes=[pltpu.VMEM((_N, _N), jnp.float32)],
        compiler_params=pltpu.CompilerParams(
            dimension_semantics=("arbitrary", "arbitrary")),
    )(x, attn, extras)
    return xm, idx.reshape(B, N)


# speedup vs baseline: 3.1896x; 3.1896x over previous
import jax
import jax.numpy as jnp
from jax.experimental import pallas as pl
from jax.experimental.pallas import tpu as pltpu

_N = 1024
_C = 192
_H = 8
_CN = 256
_K = 5
_ALPHA = 0.2
_SQRT_C = float(_C ** 0.5)


def _atm_body(x_ref, attn_ref, extras_ref, xm_ref, idx_ref, acc_ref):
    h = pl.program_id(1)

    @pl.when(h == 0)
    def _init():
        acc_ref[...] = attn_ref[0, 0]

    @pl.when(h > 0)
    def _accum():
        acc_ref[...] = acc_ref[...] + attn_ref[0, 0]

    @pl.when(h == _H - 1)
    def _compute():
        X = x_ref[0]
        A = acc_ref[...]

        n1 = jnp.sum(X * X, axis=1)
        g1 = jax.lax.dot_general(X, X, (((1,), (1,)), ((), ())),
                                 preferred_element_type=jnp.float32)
        d1 = jnp.sqrt(jnp.maximum(n1[:, None] + n1[None, :] - 2.0 * g1, 0.0))
        n2 = jnp.sum(A * A, axis=1)
        g2 = jax.lax.dot_general(A, A, (((1,), (1,)), ((), ())),
                                 preferred_element_type=jnp.float32)
        d2 = jnp.sqrt(jnp.maximum(n2[:, None] + n2[None, :] - 2.0 * g2, 0.0))
        dist = (1.0 - _ALPHA) * (d1 / _SQRT_C) + _ALPHA * (d2 / _SQRT_C)
        dist_max = jnp.max(dist, keepdims=True).reshape(1, 1)

        s = jnp.zeros((1, _N), jnp.float32)
        rem = jnp.full((1, _N), float(_K), jnp.float32)
        m = jnp.full((1, _N), -jnp.inf, jnp.float32)
        for _ in range(_K):
            cand = jnp.where(dist > m, dist, jnp.inf)
            m = jnp.min(cand, axis=0, keepdims=True)
            c = jnp.sum((dist == m).astype(jnp.float32), axis=0, keepdims=True)
            t = jnp.minimum(c, rem)
            s = s + jnp.where(t > 0.0, m * m * t, 0.0)
            rem = rem - t
        noise = extras_ref[0, 0:1, :]
        w_as = extras_ref[0, 1:2, :]
        dens = jnp.exp(-(s / float(_K))) + noise
        dens_col = dens.reshape(_N, 1)

        masked = jnp.where(dens_col > dens, dist, dist_max)
        dist_min = jnp.min(masked, axis=0, keepdims=True)

        score = dist_min * dens + w_as
        score_col = score.reshape(_N, 1)
        iota_k = jax.lax.broadcasted_iota(jnp.int32, (_N, _N), 0)
        iota_i = jax.lax.broadcasted_iota(jnp.int32, (_N, _N), 1)
        before = (score_col > score) | ((score_col == score) & (iota_k < iota_i))
        rank = jnp.sum(before.astype(jnp.float32), axis=0, keepdims=True)
        rank_col = rank.reshape(_N, 1).astype(jnp.int32)

        iota_cols = jax.lax.broadcasted_iota(jnp.int32, (_N, _CN), 1)
        onehot = (rank_col == iota_cols).astype(jnp.float32)
        dmc = jax.lax.dot_general(dist, onehot, (((1,), (0,)), ((), ())),
                                  preferred_element_type=jnp.float32,
                                  precision=jax.lax.Precision.HIGHEST)
        mn = jnp.min(dmc, axis=1, keepdims=True)
        amn = jnp.min(jnp.where(dmc == mn, iota_cols, _CN), axis=1,
                      keepdims=True)
        idx_cluster = jnp.where(rank_col < _CN, rank_col, amn)

        assign = (iota_cols == idx_cluster).astype(jnp.float32)
        counts = jnp.sum(assign, axis=0, keepdims=True)
        sums = jax.lax.dot_general(assign, X, (((0,), (0,)), ((), ())),
                                   preferred_element_type=jnp.float32,
                                   precision=jax.lax.Precision.HIGHEST)
        xm_ref[0] = sums / (counts.reshape(_CN, 1) + 1e-06)
        idx_ref[0] = idx_cluster.reshape(1, _N)


def kernel(x, attn, as_out, cluster_num):
    B, N, C = x.shape
    weight = as_out.reshape(B, -1).astype(x.dtype)
    noise = jax.random.uniform(jax.random.key(1), (B, N), dtype=x.dtype) * 1e-06
    extras = jnp.stack([noise, weight], axis=1)
    xm, idx = pl.pallas_call(
        _atm_body,
        grid=(B, _H),
        in_specs=[
            pl.BlockSpec((1, N, C), lambda b, h: (b, 0, 0)),
            pl.BlockSpec((1, 1, N, N), lambda b, h: (b, h, 0, 0)),
            pl.BlockSpec((1, 2, N), lambda b, h: (b, 0, 0)),
        ],
        out_specs=[
            pl.BlockSpec((1, _CN, C), lambda b, h: (b, 0, 0)),
            pl.BlockSpec((1, 1, N), lambda b, h: (b, 0, 0)),
        ],
        out_shape=[
            jax.ShapeDtypeStruct((B, _CN, C), x.dtype),
            jax.ShapeDtypeStruct((B, 1, N), jnp.int32),
        ],
        scratch_shapes=[pltpu.VMEM((_N, _N), jnp.float32)],
        compiler_params=pltpu.CompilerParams(
            dimension_semantics=("arbitrary", "arbitrary")),
    )(x, attn, extras)
    return xm, idx.reshape(B, N)


# speedup vs baseline: 4.0056x; 1.2558x over previous
import jax
import jax.numpy as jnp
from jax.experimental import pallas as pl
from jax.experimental.pallas import tpu as pltpu

_N = 1024
_C = 192
_H = 8
_CN = 256
_K = 5
_ALPHA = 0.2
_SQRT_C = float(_C ** 0.5)


def _atm_body(x_ref, attn_ref, extras_ref, xm_ref, idx_ref, acc_ref):
    h = pl.program_id(1)

    @pl.when(h == 0)
    def _init():
        acc_ref[...] = attn_ref[0, 0]

    @pl.when(h > 0)
    def _accum():
        acc_ref[...] = acc_ref[...] + attn_ref[0, 0]

    @pl.when(h == _H - 1)
    def _compute():
        X = x_ref[0]
        A = acc_ref[...]

        n1 = jnp.sum(X * X, axis=1)
        g1 = jax.lax.dot_general(X, X, (((1,), (1,)), ((), ())),
                                 preferred_element_type=jnp.float32)
        d1 = jnp.sqrt(jnp.maximum(n1[:, None] + n1[None, :] - 2.0 * g1, 0.0))
        n2 = jnp.sum(A * A, axis=1)
        g2 = jax.lax.dot_general(A, A, (((1,), (1,)), ((), ())),
                                 preferred_element_type=jnp.float32)
        d2 = jnp.sqrt(jnp.maximum(n2[:, None] + n2[None, :] - 2.0 * g2, 0.0))
        dist = (1.0 - _ALPHA) * (d1 / _SQRT_C) + _ALPHA * (d2 / _SQRT_C)
        dist_max = jnp.max(dist, keepdims=True).reshape(1, 1)

        s = jnp.zeros((1, _N), jnp.float32)
        rem = jnp.full((1, _N), float(_K), jnp.float32)
        m = jnp.full((1, _N), -jnp.inf, jnp.float32)
        for _ in range(_K):
            cand = jnp.where(dist > m, dist, jnp.inf)
            m = jnp.min(cand, axis=0, keepdims=True)
            c = jnp.sum((dist == m).astype(jnp.float32), axis=0, keepdims=True)
            t = jnp.minimum(c, rem)
            s = s + jnp.where(t > 0.0, m * m * t, 0.0)
            rem = rem - t
        noise = extras_ref[0, 0:1, :]
        w_as = extras_ref[0, 1:2, :]
        dens = jnp.exp(-(s / float(_K))) + noise
        dens_col = dens.reshape(_N, 1)

        masked = jnp.where(dens_col > dens, dist, dist_max)
        dist_min = jnp.min(masked, axis=0, keepdims=True)

        score = dist_min * dens + w_as
        score_col = score.reshape(_N, 1)
        iota_k = jax.lax.broadcasted_iota(jnp.int32, (_N, _N), 0)
        iota_i = jax.lax.broadcasted_iota(jnp.int32, (_N, _N), 1)
        before = (score_col > score) | ((score_col == score) & (iota_k < iota_i))
        rank = jnp.sum(before.astype(jnp.float32), axis=0, keepdims=True)
        rank_col = rank.reshape(_N, 1).astype(jnp.int32)

        is_center_row = rank < float(_CN)
        cand_d = jnp.where(is_center_row, dist, jnp.inf)
        mn = jnp.min(cand_d, axis=1, keepdims=True)
        rank_row_i = jnp.broadcast_to(rank, (1, _N))
        amn = jnp.min(jnp.where(cand_d == mn, rank_row_i, float(_CN)),
                      axis=1, keepdims=True).astype(jnp.int32)
        idx_cluster = jnp.where(rank_col < _CN, rank_col, amn)

        iota_cols = jax.lax.broadcasted_iota(jnp.int32, (_N, _CN), 1)
        assign = (iota_cols == idx_cluster).astype(jnp.float32)
        counts = jnp.sum(assign, axis=0, keepdims=True)
        sums = jax.lax.dot_general(assign, X, (((0,), (0,)), ((), ())),
                                   preferred_element_type=jnp.float32,
                                   precision=jax.lax.Precision.HIGHEST)
        xm_ref[0] = sums / (counts.reshape(_CN, 1) + 1e-06)
        idx_ref[0] = idx_cluster.reshape(1, _N)


def kernel(x, attn, as_out, cluster_num):
    B, N, C = x.shape
    weight = as_out.reshape(B, -1).astype(x.dtype)
    noise = jax.random.uniform(jax.random.key(1), (B, N), dtype=x.dtype) * 1e-06
    extras = jnp.stack([noise, weight], axis=1)
    xm, idx = pl.pallas_call(
        _atm_body,
        grid=(B, _H),
        in_specs=[
            pl.BlockSpec((1, N, C), lambda b, h: (b, 0, 0)),
            pl.BlockSpec((1, 1, N, N), lambda b, h: (b, h, 0, 0)),
            pl.BlockSpec((1, 2, N), lambda b, h: (b, 0, 0)),
        ],
        out_specs=[
            pl.BlockSpec((1, _CN, C), lambda b, h: (b, 0, 0)),
            pl.BlockSpec((1, 1, N), lambda b, h: (b, 0, 0)),
        ],
        out_shape=[
            jax.ShapeDtypeStruct((B, _CN, C), x.dtype),
            jax.ShapeDtypeStruct((B, 1, N), jnp.int32),
        ],
        scratch_shapes=[pltpu.VMEM((_N, _N), jnp.float32)],
        compiler_params=pltpu.CompilerParams(
            dimension_semantics=("arbitrary", "arbitrary")),
    )(x, attn, extras)
    return xm, idx.reshape(B, N)


# speedup vs baseline: 4.1337x; 1.0320x over previous
import jax
import jax.numpy as jnp
from jax.experimental import pallas as pl
from jax.experimental.pallas import tpu as pltpu

_N = 1024
_C = 192
_H = 8
_CN = 256
_K = 5
_ALPHA = 0.2
_SQRT_C = float(_C ** 0.5)
_CHUNK = 128


def _atm_body(x_ref, attn_ref, extras_ref, xm_ref, idx_ref, acc_ref):
    c = pl.program_id(1)
    nchunk = _N // _CHUNK

    acc_ref[pl.ds(c * _CHUNK, _CHUNK), :] = jnp.sum(attn_ref[0], axis=0)

    @pl.when(c == nchunk - 1)
    def _compute():
        X = x_ref[0]
        A = acc_ref[...]

        n1 = jnp.sum(X * X, axis=1)
        g1 = jax.lax.dot_general(X, X, (((1,), (1,)), ((), ())),
                                 preferred_element_type=jnp.float32)
        d1 = jnp.sqrt(jnp.maximum(n1[:, None] + n1[None, :] - 2.0 * g1, 0.0))
        n2 = jnp.sum(A * A, axis=1)
        g2 = jax.lax.dot_general(A, A, (((1,), (1,)), ((), ())),
                                 preferred_element_type=jnp.float32)
        d2 = jnp.sqrt(jnp.maximum(n2[:, None] + n2[None, :] - 2.0 * g2, 0.0))
        dist = (1.0 - _ALPHA) * (d1 / _SQRT_C) + _ALPHA * (d2 / _SQRT_C)
        dist_max = jnp.max(dist, keepdims=True).reshape(1, 1)

        s = jnp.zeros((1, _N), jnp.float32)
        rem = jnp.full((1, _N), float(_K), jnp.float32)
        m = None
        for lvl in range(_K):
            if lvl == 0:
                m = jnp.min(dist, axis=0, keepdims=True)
            else:
                cand = jnp.where(dist > m, dist, jnp.inf)
                m = jnp.min(cand, axis=0, keepdims=True)
            cnt = jnp.sum((dist == m).astype(jnp.float32), axis=0,
                          keepdims=True)
            t = jnp.minimum(cnt, rem)
            s = s + jnp.where(t > 0.0, m * m * t, 0.0)
            rem = rem - t
        noise = extras_ref[0, 0:1, :]
        w_as = extras_ref[0, 1:2, :]
        dens = jnp.exp(-(s / float(_K))) + noise
        dens_col = dens.reshape(_N, 1)

        masked = jnp.where(dens_col > dens, dist, dist_max)
        dist_min = jnp.min(masked, axis=0, keepdims=True)

        score = dist_min * dens + w_as
        score_col = score.reshape(_N, 1)
        iota_k = jax.lax.broadcasted_iota(jnp.int32, (_N, _N), 0)
        iota_i = jax.lax.broadcasted_iota(jnp.int32, (_N, _N), 1)
        before = (score_col > score) | ((score_col == score) & (iota_k < iota_i))
        rank = jnp.sum(before.astype(jnp.float32), axis=0, keepdims=True)
        rank_col = rank.reshape(_N, 1).astype(jnp.int32)

        is_center_row = rank < float(_CN)
        cand_d = jnp.where(is_center_row, dist, jnp.inf)
        mn = jnp.min(cand_d, axis=1, keepdims=True)
        rank_row_i = jnp.broadcast_to(rank, (1, _N))
        amn = jnp.min(jnp.where(cand_d == mn, rank_row_i, float(_CN)),
                      axis=1, keepdims=True).astype(jnp.int32)
        idx_cluster = jnp.where(rank_col < _CN, rank_col, amn)

        iota_cols = jax.lax.broadcasted_iota(jnp.int32, (_N, _CN), 1)
        assign = (iota_cols == idx_cluster).astype(jnp.float32)
        counts = jnp.sum(assign, axis=0, keepdims=True)
        sums = jax.lax.dot_general(assign, X, (((0,), (0,)), ((), ())),
                                   preferred_element_type=jnp.float32,
                                   precision=jax.lax.Precision.HIGHEST)
        xm_ref[0] = sums / (counts.reshape(_CN, 1) + 1e-06)
        idx_ref[0] = idx_cluster.reshape(1, _N)


def kernel(x, attn, as_out, cluster_num):
    B, N, C = x.shape
    weight = as_out.reshape(B, -1).astype(x.dtype)
    noise = jax.random.uniform(jax.random.key(1), (B, N), dtype=x.dtype) * 1e-06
    extras = jnp.stack([noise, weight], axis=1)
    xm, idx = pl.pallas_call(
        _atm_body,
        grid=(B, N // _CHUNK),
        in_specs=[
            pl.BlockSpec((1, N, C), lambda b, c: (b, 0, 0)),
            pl.BlockSpec((1, _H, _CHUNK, N), lambda b, c: (b, 0, c, 0)),
            pl.BlockSpec((1, 2, N), lambda b, c: (b, 0, 0)),
        ],
        out_specs=[
            pl.BlockSpec((1, _CN, C), lambda b, c: (b, 0, 0)),
            pl.BlockSpec((1, 1, N), lambda b, c: (b, 0, 0)),
        ],
        out_shape=[
            jax.ShapeDtypeStruct((B, _CN, C), x.dtype),
            jax.ShapeDtypeStruct((B, 1, N), jnp.int32),
        ],
        scratch_shapes=[pltpu.VMEM((_N, _N), jnp.float32)],
        compiler_params=pltpu.CompilerParams(
            dimension_semantics=("arbitrary", "arbitrary")),
    )(x, attn, extras)
    return xm, idx.reshape(B, N)


# speedup vs baseline: 4.5815x; 1.1083x over previous
import jax
import jax.numpy as jnp
from jax.experimental import pallas as pl
from jax.experimental.pallas import tpu as pltpu

_N = 1024
_C = 192
_H = 8
_CN = 256
_K = 5
_ALPHA = 0.2
_SQRT_C = float(_C ** 0.5)
_CHUNK = 128


_NBUF = 4


def _atm_body(x_ref, attn_ref, extras_ref, xm_ref, idx_ref,
              acc_ref, buf_ref, sem_ref):
    b = pl.program_id(0)
    c = pl.program_id(1)
    nchunk = _N // _CHUNK
    nb = pl.num_programs(0)
    t = b * nchunk + c

    def _issue(k):
        bb = k // nchunk
        cc = k % nchunk

        @pl.when(k < nb * nchunk)
        def _():
            pltpu.make_async_copy(
                attn_ref.at[bb, :, pl.ds(cc * _CHUNK, _CHUNK), :],
                buf_ref.at[k % _NBUF],
                sem_ref.at[k % _NBUF],
            ).start()

    @pl.when(t == 0)
    def _prologue():
        for k in range(_NBUF):
            _issue(k)

    pltpu.make_async_copy(
        attn_ref.at[b, :, pl.ds(c * _CHUNK, _CHUNK), :],
        buf_ref.at[t % _NBUF],
        sem_ref.at[t % _NBUF],
    ).wait()
    acc_ref[pl.ds(c * _CHUNK, _CHUNK), :] = jnp.sum(buf_ref[t % _NBUF], axis=0)
    _issue(t + _NBUF)

    @pl.when(c == nchunk - 1)
    def _compute():
        X = x_ref[0]
        A = acc_ref[...]

        n1 = jnp.sum(X * X, axis=1)
        g1 = jax.lax.dot_general(X, X, (((1,), (1,)), ((), ())),
                                 preferred_element_type=jnp.float32)
        d1 = jnp.sqrt(jnp.maximum(n1[:, None] + n1[None, :] - 2.0 * g1, 0.0))
        n2 = jnp.sum(A * A, axis=1)
        g2 = jax.lax.dot_general(A, A, (((1,), (1,)), ((), ())),
                                 preferred_element_type=jnp.float32)
        d2 = jnp.sqrt(jnp.maximum(n2[:, None] + n2[None, :] - 2.0 * g2, 0.0))
        dist = (1.0 - _ALPHA) * (d1 / _SQRT_C) + _ALPHA * (d2 / _SQRT_C)
        dist_max = jnp.max(dist, keepdims=True).reshape(1, 1)

        s = jnp.zeros((1, _N), jnp.float32)
        rem = jnp.full((1, _N), float(_K), jnp.float32)
        m = None
        for lvl in range(_K):
            if lvl == 0:
                m = jnp.min(dist, axis=0, keepdims=True)
            else:
                cand = jnp.where(dist > m, dist, jnp.inf)
                m = jnp.min(cand, axis=0, keepdims=True)
            cnt = jnp.sum((dist == m).astype(jnp.float32), axis=0,
                          keepdims=True)
            t = jnp.minimum(cnt, rem)
            s = s + jnp.where(t > 0.0, m * m * t, 0.0)
            rem = rem - t
        noise = extras_ref[0, 0:1, :]
        w_as = extras_ref[0, 1:2, :]
        dens = jnp.exp(-(s / float(_K))) + noise
        dens_col = dens.reshape(_N, 1)

        masked = jnp.where(dens_col > dens, dist, dist_max)
        dist_min = jnp.min(masked, axis=0, keepdims=True)

        score = dist_min * dens + w_as
        score_col = score.reshape(_N, 1)
        iota_k = jax.lax.broadcasted_iota(jnp.int32, (_N, _N), 0)
        iota_i = jax.lax.broadcasted_iota(jnp.int32, (_N, _N), 1)
        before = (score_col > score) | ((score_col == score) & (iota_k < iota_i))
        rank = jnp.sum(before.astype(jnp.float32), axis=0, keepdims=True)
        rank_col = rank.reshape(_N, 1).astype(jnp.int32)

        is_center_row = rank < float(_CN)
        cand_d = jnp.where(is_center_row, dist, jnp.inf)
        mn = jnp.min(cand_d, axis=1, keepdims=True)
        rank_row_i = jnp.broadcast_to(rank, (1, _N))
        amn = jnp.min(jnp.where(cand_d == mn, rank_row_i, float(_CN)),
                      axis=1, keepdims=True).astype(jnp.int32)
        idx_cluster = jnp.where(rank_col < _CN, rank_col, amn)

        iota_cols = jax.lax.broadcasted_iota(jnp.int32, (_N, _CN), 1)
        assign = (iota_cols == idx_cluster).astype(jnp.float32)
        counts = jnp.sum(assign, axis=0, keepdims=True)
        sums = jax.lax.dot_general(assign, X, (((0,), (0,)), ((), ())),
                                   preferred_element_type=jnp.float32,
                                   precision=jax.lax.Precision.HIGHEST)
        xm_ref[0] = sums / (counts.reshape(_CN, 1) + 1e-06)
        idx_ref[0] = idx_cluster.reshape(1, _N)


def kernel(x, attn, as_out, cluster_num):
    B, N, C = x.shape
    weight = as_out.reshape(B, -1).astype(x.dtype)
    noise = jax.random.uniform(jax.random.key(1), (B, N), dtype=x.dtype) * 1e-06
    extras = jnp.stack([noise, weight], axis=1)
    xm, idx = pl.pallas_call(
        _atm_body,
        grid=(B, N // _CHUNK),
        in_specs=[
            pl.BlockSpec((1, N, C), lambda b, c: (b, 0, 0)),
            pl.BlockSpec(memory_space=pltpu.MemorySpace.HBM),
            pl.BlockSpec((1, 2, N), lambda b, c: (b, 0, 0)),
        ],
        out_specs=[
            pl.BlockSpec((1, _CN, C), lambda b, c: (b, 0, 0)),
            pl.BlockSpec((1, 1, N), lambda b, c: (b, 0, 0)),
        ],
        out_shape=[
            jax.ShapeDtypeStruct((B, _CN, C), x.dtype),
            jax.ShapeDtypeStruct((B, 1, N), jnp.int32),
        ],
        scratch_shapes=[
            pltpu.VMEM((_N, _N), jnp.float32),
            pltpu.VMEM((_NBUF, _H, _CHUNK, _N), jnp.float32),
            pltpu.SemaphoreType.DMA((_NBUF,)),
        ],
        compiler_params=pltpu.CompilerParams(
            dimension_semantics=("arbitrary", "arbitrary")),
    )(x, attn, extras)
    return xm, idx.reshape(B, N)


# speedup vs baseline: 4.9483x; 1.0801x over previous
import jax
import jax.numpy as jnp
from jax.experimental import pallas as pl
from jax.experimental.pallas import tpu as pltpu

_N = 1024
_C = 192
_H = 8
_CN = 256
_K = 5
_ALPHA = 0.2
_SQRT_C = float(_C ** 0.5)
_CHUNK = 128


_NBUF = 6


def _atm_body(x_ref, attn_ref, extras_ref, xm_ref, idx_ref,
              acc_ref, buf_ref, sem_ref):
    b = pl.program_id(0)
    c = pl.program_id(1)
    nchunk = _N // _CHUNK
    nb = pl.num_programs(0)
    t = b * nchunk + c

    def _issue(k):
        bb = k // nchunk
        cc = k % nchunk

        @pl.when(k < nb * nchunk)
        def _():
            pltpu.make_async_copy(
                attn_ref.at[bb, :, pl.ds(cc * _CHUNK, _CHUNK), :],
                buf_ref.at[k % _NBUF],
                sem_ref.at[k % _NBUF],
            ).start()

    @pl.when(t == 0)
    def _prologue():
        for k in range(_NBUF):
            _issue(k)

    pltpu.make_async_copy(
        attn_ref.at[b, :, pl.ds(c * _CHUNK, _CHUNK), :],
        buf_ref.at[t % _NBUF],
        sem_ref.at[t % _NBUF],
    ).wait()
    acc_ref[pl.ds(c * _CHUNK, _CHUNK), :] = jnp.sum(buf_ref[t % _NBUF], axis=0)
    _issue(t + _NBUF)

    @pl.when(c == nchunk - 1)
    def _compute():
        X = x_ref[0]
        A = acc_ref[...]

        n1 = jnp.sum(X * X, axis=1)
        g1 = jax.lax.dot_general(X, X, (((1,), (1,)), ((), ())),
                                 preferred_element_type=jnp.float32)
        d1 = jnp.sqrt(jnp.maximum(n1[:, None] + n1[None, :] - 2.0 * g1, 0.0))
        n2 = jnp.sum(A * A, axis=1)
        g2 = jax.lax.dot_general(A, A, (((1,), (1,)), ((), ())),
                                 preferred_element_type=jnp.float32)
        d2 = jnp.sqrt(jnp.maximum(n2[:, None] + n2[None, :] - 2.0 * g2, 0.0))
        dist = (1.0 - _ALPHA) * (d1 / _SQRT_C) + _ALPHA * (d2 / _SQRT_C)
        dist_max = jnp.max(dist, keepdims=True).reshape(1, 1)

        s = jnp.zeros((1, _N), jnp.float32)
        rem = jnp.full((1, _N), float(_K), jnp.float32)
        m = None
        for lvl in range(_K):
            if lvl == 0:
                m = jnp.min(dist, axis=0, keepdims=True)
            else:
                cand = jnp.where(dist > m, dist, jnp.inf)
                m = jnp.min(cand, axis=0, keepdims=True)
            cnt = jnp.sum((dist == m).astype(jnp.float32), axis=0,
                          keepdims=True)
            t = jnp.minimum(cnt, rem)
            s = s + jnp.where(t > 0.0, m * m * t, 0.0)
            rem = rem - t
        noise = extras_ref[0, 0:1, :]
        w_as = extras_ref[0, 1:2, :]
        dens = jnp.exp(-(s / float(_K))) + noise
        dens_col = dens.reshape(_N, 1)

        masked = jnp.where(dens_col > dens, dist, dist_max)
        dist_min = jnp.min(masked, axis=0, keepdims=True)

        score = dist_min * dens + w_as
        score_col = score.reshape(_N, 1)
        iota_k = jax.lax.broadcasted_iota(jnp.int32, (_N, _N), 0)
        iota_i = jax.lax.broadcasted_iota(jnp.int32, (_N, _N), 1)
        before = (score_col > score) | ((score_col == score) & (iota_k < iota_i))
        rank = jnp.sum(before.astype(jnp.float32), axis=0, keepdims=True)
        rank_col = rank.reshape(_N, 1).astype(jnp.int32)

        is_center_row = rank < float(_CN)
        cand_d = jnp.where(is_center_row, dist, jnp.inf)
        mn = jnp.min(cand_d, axis=1, keepdims=True)
        rank_row_i = jnp.broadcast_to(rank, (1, _N))
        amn = jnp.min(jnp.where(cand_d == mn, rank_row_i, float(_CN)),
                      axis=1, keepdims=True).astype(jnp.int32)
        idx_cluster = jnp.where(rank_col < _CN, rank_col, amn)

        iota_cols = jax.lax.broadcasted_iota(jnp.int32, (_N, _CN), 1)
        assign = (iota_cols == idx_cluster).astype(jnp.float32)
        counts = jnp.sum(assign, axis=0, keepdims=True)
        sums = jax.lax.dot_general(assign, X, (((0,), (0,)), ((), ())),
                                   preferred_element_type=jnp.float32,
                                   precision=jax.lax.Precision.HIGHEST)
        xm_ref[0] = sums / (counts.reshape(_CN, 1) + 1e-06)
        idx_ref[0] = idx_cluster.reshape(1, _N)


def kernel(x, attn, as_out, cluster_num):
    B, N, C = x.shape
    weight = as_out.reshape(B, -1).astype(x.dtype)
    noise = jax.random.uniform(jax.random.key(1), (B, N), dtype=x.dtype) * 1e-06
    extras = jnp.stack([noise, weight], axis=1)
    xm, idx = pl.pallas_call(
        _atm_body,
        grid=(B, N // _CHUNK),
        in_specs=[
            pl.BlockSpec((1, N, C), lambda b, c: (b, 0, 0)),
            pl.BlockSpec(memory_space=pltpu.MemorySpace.HBM),
            pl.BlockSpec((1, 2, N), lambda b, c: (b, 0, 0)),
        ],
        out_specs=[
            pl.BlockSpec((1, _CN, C), lambda b, c: (b, 0, 0)),
            pl.BlockSpec((1, 1, N), lambda b, c: (b, 0, 0)),
        ],
        out_shape=[
            jax.ShapeDtypeStruct((B, _CN, C), x.dtype),
            jax.ShapeDtypeStruct((B, 1, N), jnp.int32),
        ],
        scratch_shapes=[
            pltpu.VMEM((_N, _N), jnp.float32),
            pltpu.VMEM((_NBUF, _H, _CHUNK, _N), jnp.float32),
            pltpu.SemaphoreType.DMA((_NBUF,)),
        ],
        compiler_params=pltpu.CompilerParams(
            dimension_semantics=("arbitrary", "arbitrary")),
    )(x, attn, extras)
    return xm, idx.reshape(B, N)


# speedup vs baseline: 5.2096x; 1.0528x over previous
import jax
import jax.numpy as jnp
from jax.experimental import pallas as pl
from jax.experimental.pallas import tpu as pltpu

_N = 1024
_C = 192
_H = 8
_CN = 256
_K = 5
_ALPHA = 0.2
_SQRT_C = float(_C ** 0.5)
_CHUNK = 128


_NBUF = 8


def _atm_body(x_ref, attn_ref, extras_ref, xm_ref, idx_ref,
              acc_ref, buf_ref, sem_ref):
    b = pl.program_id(0)
    c = pl.program_id(1)
    nchunk = _N // _CHUNK
    nb = pl.num_programs(0)
    t = b * nchunk + c

    def _issue(k):
        bb = k // nchunk
        cc = k % nchunk

        @pl.when(k < nb * nchunk)
        def _():
            pltpu.make_async_copy(
                attn_ref.at[bb, :, pl.ds(cc * _CHUNK, _CHUNK), :],
                buf_ref.at[k % _NBUF],
                sem_ref.at[k % _NBUF],
            ).start()

    @pl.when(t == 0)
    def _prologue():
        for k in range(_NBUF):
            _issue(k)

    pltpu.make_async_copy(
        attn_ref.at[b, :, pl.ds(c * _CHUNK, _CHUNK), :],
        buf_ref.at[t % _NBUF],
        sem_ref.at[t % _NBUF],
    ).wait()
    acc_ref[pl.ds(c * _CHUNK, _CHUNK), :] = jnp.sum(buf_ref[t % _NBUF], axis=0)
    _issue(t + _NBUF)

    @pl.when(c == nchunk - 1)
    def _compute():
        X = x_ref[0]
        A = acc_ref[...]

        n1 = jnp.sum(X * X, axis=1)
        g1 = jax.lax.dot_general(X, X, (((1,), (1,)), ((), ())),
                                 preferred_element_type=jnp.float32)
        d1 = jnp.sqrt(jnp.maximum(n1[:, None] + n1[None, :] - 2.0 * g1, 0.0))
        n2 = jnp.sum(A * A, axis=1)
        g2 = jax.lax.dot_general(A, A, (((1,), (1,)), ((), ())),
                                 preferred_element_type=jnp.float32)
        d2 = jnp.sqrt(jnp.maximum(n2[:, None] + n2[None, :] - 2.0 * g2, 0.0))
        dist = (1.0 - _ALPHA) * (d1 / _SQRT_C) + _ALPHA * (d2 / _SQRT_C)
        dist_max = jnp.max(dist, keepdims=True).reshape(1, 1)

        s = jnp.zeros((1, _N), jnp.float32)
        rem = jnp.full((1, _N), float(_K), jnp.float32)
        m = None
        for lvl in range(_K):
            if lvl == 0:
                m = jnp.min(dist, axis=0, keepdims=True)
            else:
                cand = jnp.where(dist > m, dist, jnp.inf)
                m = jnp.min(cand, axis=0, keepdims=True)
            cnt = jnp.sum((dist == m).astype(jnp.float32), axis=0,
                          keepdims=True)
            t = jnp.minimum(cnt, rem)
            s = s + jnp.where(t > 0.0, m * m * t, 0.0)
            rem = rem - t
        noise = extras_ref[0, 0:1, :]
        w_as = extras_ref[0, 1:2, :]
        dens = jnp.exp(-(s / float(_K))) + noise
        dens_col = dens.reshape(_N, 1)

        masked = jnp.where(dens_col > dens, dist, dist_max)
        dist_min = jnp.min(masked, axis=0, keepdims=True)

        score = dist_min * dens + w_as
        score_col = score.reshape(_N, 1)
        iota_k = jax.lax.broadcasted_iota(jnp.int32, (_N, _N), 0)
        iota_i = jax.lax.broadcasted_iota(jnp.int32, (_N, _N), 1)
        before = (score_col > score) | ((score_col == score) & (iota_k < iota_i))
        rank = jnp.sum(before.astype(jnp.float32), axis=0, keepdims=True)
        rank_col = rank.reshape(_N, 1).astype(jnp.int32)

        is_center_row = rank < float(_CN)
        cand_d = jnp.where(is_center_row, dist, jnp.inf)
        mn = jnp.min(cand_d, axis=1, keepdims=True)
        rank_row_i = jnp.broadcast_to(rank, (1, _N))
        amn = jnp.min(jnp.where(cand_d == mn, rank_row_i, float(_CN)),
                      axis=1, keepdims=True).astype(jnp.int32)
        idx_cluster = jnp.where(rank_col < _CN, rank_col, amn)

        iota_cols = jax.lax.broadcasted_iota(jnp.int32, (_N, _CN), 1)
        assign = (iota_cols == idx_cluster).astype(jnp.float32)
        counts = jnp.sum(assign, axis=0, keepdims=True)
        sums = jax.lax.dot_general(assign, X, (((0,), (0,)), ((), ())),
                                   preferred_element_type=jnp.float32,
                                   precision=jax.lax.Precision.HIGHEST)
        xm_ref[0] = sums / (counts.reshape(_CN, 1) + 1e-06)
        idx_ref[0] = idx_cluster.reshape(1, _N)


def kernel(x, attn, as_out, cluster_num):
    B, N, C = x.shape
    weight = as_out.reshape(B, -1).astype(x.dtype)
    noise = jax.random.uniform(jax.random.key(1), (B, N), dtype=x.dtype) * 1e-06
    extras = jnp.stack([noise, weight], axis=1)
    xm, idx = pl.pallas_call(
        _atm_body,
        grid=(B, N // _CHUNK),
        in_specs=[
            pl.BlockSpec((1, N, C), lambda b, c: (b, 0, 0)),
            pl.BlockSpec(memory_space=pltpu.MemorySpace.HBM),
            pl.BlockSpec((1, 2, N), lambda b, c: (b, 0, 0)),
        ],
        out_specs=[
            pl.BlockSpec((1, _CN, C), lambda b, c: (b, 0, 0)),
            pl.BlockSpec((1, 1, N), lambda b, c: (b, 0, 0)),
        ],
        out_shape=[
            jax.ShapeDtypeStruct((B, _CN, C), x.dtype),
            jax.ShapeDtypeStruct((B, 1, N), jnp.int32),
        ],
        scratch_shapes=[
            pltpu.VMEM((_N, _N), jnp.float32),
            pltpu.VMEM((_NBUF, _H, _CHUNK, _N), jnp.float32),
            pltpu.SemaphoreType.DMA((_NBUF,)),
        ],
        compiler_params=pltpu.CompilerParams(
            dimension_semantics=("arbitrary", "arbitrary")),
    )(x, attn, extras)
    return xm, idx.reshape(B, N)
